# baseline (speedup 1.0000x reference)
import functools

import jax
import jax.numpy as jnp
from jax import lax
from jax.experimental import pallas as pl
from jax.experimental.pallas import tpu as pltpu
from jax.experimental.pallas import tpu_sc as plsc

NC = 2
NS = 16
NW = NC * NS
CHUNK = 128
F = 16
TW = (F + 1) * F


def _taug_rows(base, nrows, feat_hbm, taug_out, fbuf, wcv, obuf):
    pltpu.sync_copy(feat_hbm.at[pl.ds(base, nrows)], fbuf.at[pl.ds(0, nrows)])
    for g in range(F + 1):
        wg = [wcv[i, pl.ds(g * F, F)] for i in range(F)]

        @pl.loop(0, nrows)
        def _(n):
            f_row = fbuf[n, :]
            acc = [None, None, None, None]
            for i in range(F):
                t = f_row[i] * wg[i]
                k = i % 4
                acc[k] = t if acc[k] is None else acc[k] + t
            obuf[n, pl.ds(g * F, F)] = (acc[0] + acc[1]) + (acc[2] + acc[3])

    pltpu.sync_copy(obuf.at[pl.ds(0, nrows)], taug_out.at[pl.ds(base, nrows)])


def _taug_body(npw, tfullw, tlastn, feat_hbm, wc_hbm, taug_out,
               fbuf, wcv, obuf):
    c = lax.axis_index("c")
    s = lax.axis_index("s")
    wid = s * NC + c
    pltpu.sync_copy(wc_hbm, wcv)

    @pl.when(wid < tfullw)
    def _():
        _taug_rows(wid * npw, npw, feat_hbm, taug_out, fbuf, wcv, obuf)

    if tlastn:
        @pl.when(wid == tfullw)
        def _():
            _taug_rows(wid * npw, tlastn, feat_hbm, taug_out, fbuf, wcv, obuf)


def _sc_taug(feat, Wc, n_nodes):
    npw = -(-n_nodes // NW)
    npw = -(-npw // 8) * 8
    tfullw = n_nodes // npw
    tlastn = n_nodes - tfullw * npw
    mesh = plsc.VectorSubcoreMesh(core_axis_name="c", subcore_axis_name="s")
    return pl.kernel(
        functools.partial(_taug_body, npw, tfullw, tlastn),
        out_type=jax.ShapeDtypeStruct((n_nodes, TW), jnp.float32),
        mesh=mesh,
        scratch_types=[
            pltpu.VMEM((npw, F), jnp.float32),
            pltpu.VMEM((F, TW), jnp.float32),
            pltpu.VMEM((npw, TW), jnp.float32),
        ],
        compiler_params=pltpu.CompilerParams(use_tc_tiling_on_sc=False),
    )(feat, Wc)


def _compute_chunk(tbuf, efv, msgv):
    @pl.loop(0, CHUNK)
    def _(e):
        ef_row = efv[e, :]
        acc = [tbuf[e, pl.ds(F * F, F)], None, None, None]
        for d in range(F):
            t = ef_row[d] * tbuf[e, pl.ds(d * F, F)]
            k = d % 4
            acc[k] = t if acc[k] is None else acc[k] + t
        msgv[e, :] = (acc[0] + acc[1]) + (acc[2] + acc[3])


def _edge_chunks(base, nj, sidx, didx, taug_hbm, ef_hbm, acc_sh, deg_sh,
                 tbufs, efvs, msgs, onesv, gsems, esems, ssems, dsem):
    def fire(jj, b):
        pltpu.async_copy(ef_hbm.at[pl.ds(base + jj * CHUNK, CHUNK)],
                         efvs[b], esems[b])
        pltpu.async_copy(taug_hbm.at[sidx.at[jj]], tbufs[b], gsems[b])

    fire(0, 0)
    if nj > 1:
        fire(1, 1)

    @pl.loop(0, nj, step=2)
    def _(j):
        for b in range(2):
            jj = j + b

            @pl.when(jj < nj)
            def _():
                pltpu.make_async_copy(
                    ef_hbm.at[pl.ds(base + jj * CHUNK, CHUNK)],
                    efvs[b], esems[b]).wait()
                pltpu.make_async_copy(
                    taug_hbm.at[sidx.at[jj]], tbufs[b], gsems[b]).wait()

                @pl.when(jj >= 2)
                def _():
                    pltpu.make_async_copy(
                        msgs[b], acc_sh.at[didx.at[jj]], ssems[b]).wait()

                _compute_chunk(tbufs[b], efvs[b], msgs[b])

                @pl.when(jj + 2 < nj)
                def _():
                    fire(jj + 2, b)

                pltpu.async_copy(msgs[b], acc_sh.at[didx.at[jj]], ssems[b],
                                 add=True)
                pltpu.async_copy(onesv, deg_sh.at[didx.at[jj]], dsem,
                                 add=True)

    for k in range(min(2, nj)):
        b = (nj - 1 - k) % 2
        pltpu.make_async_copy(msgs[b], acc_sh.at[didx.at[0]], ssems[b]).wait()

    @pl.loop(0, nj)
    def _(j):
        pltpu.make_async_copy(onesv, deg_sh.at[didx.at[0]], dsem).wait()


def _fused_body(kj, fullw, lastn, epw, n_acc, rps,
                taug_hbm, ef_hbm, src2_hbm, dst2_hbm, ones_hbm, zeros_hbm,
                acc_out, deg_out,
                acc_sh, deg_sh, sidx, didx, tb0, tb1, ef0, ef1, ms0, ms1,
                onesv, outv, gs0, gs1, es0, es1, ss0, ss1, dsem):
    c = lax.axis_index("c")
    s = lax.axis_index("s")
    wid = s * NC + c
    base = wid * epw
    tbufs, efvs, msgs = (tb0, tb1), (ef0, ef1), (ms0, ms1)
    gsems, esems, ssems = (gs0, gs1), (es0, es1), (ss0, ss1)
    pltpu.sync_copy(zeros_hbm, acc_sh.at[pl.ds(s * rps, rps)])
    pltpu.sync_copy(zeros_hbm, deg_sh.at[pl.ds(s * rps, rps)])
    pltpu.sync_copy(ones_hbm, onesv)
    plsc.subcore_barrier()

    @pl.when(wid < fullw)
    def _():
        pltpu.sync_copy(src2_hbm.at[pl.ds(wid * kj, kj)], sidx)
        pltpu.sync_copy(dst2_hbm.at[pl.ds(wid * kj, kj)], didx)
        _edge_chunks(base, kj, sidx, didx, taug_hbm, ef_hbm, acc_sh, deg_sh,
                     tbufs, efvs, msgs, onesv, gsems, esems, ssems, dsem)

    if lastn:
        @pl.when(wid == fullw)
        def _():
            pltpu.sync_copy(src2_hbm.at[pl.ds(wid * kj, lastn)],
                            sidx.at[pl.ds(0, lastn)])
            pltpu.sync_copy(dst2_hbm.at[pl.ds(wid * kj, lastn)],
                            didx.at[pl.ds(0, lastn)])
            _edge_chunks(base, lastn, sidx, didx, taug_hbm, ef_hbm, acc_sh,
                         deg_sh, tbufs, efvs, msgs, onesv, gsems, esems,
                         ssems, dsem)

    plsc.subcore_barrier()
    pltpu.sync_copy(acc_sh.at[pl.ds(s * rps, rps)], outv)
    pltpu.sync_copy(outv, acc_out.at[c, pl.ds(s * rps, rps)])
    pltpu.sync_copy(deg_sh.at[pl.ds(s * rps, rps)], outv)
    pltpu.sync_copy(outv, deg_out.at[c, pl.ds(s * rps, rps)])


def _sc_fused(taug, efeat, src2, dst2, n_acc, kj, fullw, lastn):
    epw = kj * CHUNK
    rps = n_acc // NS
    mesh = plsc.VectorSubcoreMesh(core_axis_name="c", subcore_axis_name="s")
    ones = jnp.ones((CHUNK, F), jnp.float32)
    zeros = jnp.zeros((rps, F), jnp.float32)
    out_sds = jax.ShapeDtypeStruct((NC, n_acc, F), jnp.float32)
    return pl.kernel(
        functools.partial(_fused_body, kj, fullw, lastn, epw, n_acc, rps),
        out_type=(out_sds, out_sds),
        mesh=mesh,
        scratch_types=[
            pltpu.VMEM_SHARED((n_acc, F), jnp.float32),
            pltpu.VMEM_SHARED((n_acc, F), jnp.float32),
            pltpu.VMEM((kj, CHUNK), jnp.int32),
            pltpu.VMEM((kj, CHUNK), jnp.int32),
            pltpu.VMEM((CHUNK, TW), jnp.float32),
            pltpu.VMEM((CHUNK, TW), jnp.float32),
            pltpu.VMEM((CHUNK, F), jnp.float32),
            pltpu.VMEM((CHUNK, F), jnp.float32),
            pltpu.VMEM((CHUNK, F), jnp.float32),
            pltpu.VMEM((CHUNK, F), jnp.float32),
            pltpu.VMEM((CHUNK, F), jnp.float32),
            pltpu.VMEM((rps, F), jnp.float32),
            pltpu.SemaphoreType.DMA,
            pltpu.SemaphoreType.DMA,
            pltpu.SemaphoreType.DMA,
            pltpu.SemaphoreType.DMA,
            pltpu.SemaphoreType.DMA,
            pltpu.SemaphoreType.DMA,
            pltpu.SemaphoreType.DMA,
        ],
        compiler_params=pltpu.CompilerParams(use_tc_tiling_on_sc=False),
    )(taug, efeat, src2, dst2, ones, zeros)


def _final_body(n_nodes, a_ref, d_ref, m_ref, bias_ref, g_ref, bt_ref, o_ref):
    acc = a_ref[0] + a_ref[1]
    deg = jnp.maximum(d_ref[0] + d_ref[1], 1.0)
    rst = acc / deg + bias_ref[...]
    ssum = jnp.sum(rst, axis=0, keepdims=True)
    ssq = jnp.sum(rst * rst, axis=0, keepdims=True)
    mean = jnp.dot(ssum, m_ref[...], preferred_element_type=jnp.float32)
    ex2 = jnp.dot(ssq, m_ref[...], preferred_element_type=jnp.float32)
    var = ex2 - mean * mean
    inv = lax.rsqrt(var + 1e-5)
    o_ref[...] = (rst - mean) * inv * g_ref[...] + bt_ref[...]


def _tc_final(accf, degf, bias, gamma, beta, n_nodes, rows):
    lanes = 128
    per = lanes // F
    mf = ((jnp.arange(lanes)[:, None] % F)
          == (jnp.arange(lanes)[None, :] % F)).astype(jnp.float32) / n_nodes
    biasf = jnp.tile(bias, per)[None, :]
    gammaf = jnp.tile(gamma, per)[None, :]
    betaf = jnp.tile(beta, per)[None, :]
    return pl.pallas_call(
        functools.partial(_final_body, n_nodes),
        out_shape=jax.ShapeDtypeStruct((rows, lanes), jnp.float32),
    )(accf, degf, mf, biasf, gammaf, betaf)


def kernel(feat, efeat, W_e, b_e, bias, gamma, beta, edge_index):
    n_nodes, in_f = feat.shape
    n_edges = edge_index.shape[1]
    out_f = bias.shape[0]

    src = edge_index[0]
    dst = edge_index[1]
    n_acc = -(-(n_nodes + 1) // (NS * 8)) * (NS * 8)
    trash = n_nodes

    if n_edges % CHUNK:
        pad = CHUNK - n_edges % CHUNK
        src = jnp.concatenate([src, jnp.zeros((pad,), jnp.int32)])
        dst = jnp.concatenate([dst, jnp.full((pad,), trash, jnp.int32)])
        efeat = jnp.concatenate([efeat, jnp.zeros((pad, in_f), efeat.dtype)])
        n_edges += pad
    nchunk = n_edges // CHUNK
    kj = -(-nchunk // NW)
    fullw = nchunk // kj
    lastn = nchunk - fullw * kj

    src2 = src.reshape(nchunk, CHUNK)
    dst2 = dst.reshape(nchunk, CHUNK)

    W2 = W_e.reshape(in_f, in_f, out_f).transpose(1, 0, 2).reshape(
        in_f, in_f * out_f)
    B = b_e.reshape(in_f, out_f)
    Wc = jnp.concatenate([W2, B], axis=1)
    taug = _sc_taug(feat, Wc, n_nodes)

    acc2, deg2 = _sc_fused(taug, efeat, src2, dst2, n_acc, kj, fullw, lastn)

    rows = n_nodes * out_f // 128
    accf = acc2[:, :n_nodes, :].reshape(NC, rows, 128)
    degf = deg2[:, :n_nodes, :].reshape(NC, rows, 128)
    outf = _tc_final(accf, degf, bias, gamma, beta, n_nodes, rows)
    return outf.reshape(n_nodes, out_f)


# speedup vs baseline: 1.3866x; 1.3866x over previous
import functools

import jax
import jax.numpy as jnp
from jax import lax
from jax.experimental import pallas as pl
from jax.experimental.pallas import tpu as pltpu
from jax.experimental.pallas import tpu_sc as plsc

NC = 2
NS = 16
NW = NC * NS
CHUNK = 128
F = 16
TW = (F + 1) * F


def _pre_body(f_ref, w2_ref, b_ref, o_ref):
    f = f_ref[...]
    t1 = jnp.dot(f, w2_ref[...], preferred_element_type=jnp.float32)
    t2 = jnp.dot(f, b_ref[...], preferred_element_type=jnp.float32)
    o_ref[...] = jnp.concatenate([t1, t2], axis=1)


def _tc_pre(feat, W2, B, n_nodes, blk):
    grid = n_nodes // blk
    return pl.pallas_call(
        _pre_body,
        grid=(grid,),
        in_specs=[
            pl.BlockSpec((blk, F), lambda i: (i, 0)),
            pl.BlockSpec((F, F * F), lambda i: (0, 0)),
            pl.BlockSpec((F, F), lambda i: (0, 0)),
        ],
        out_specs=pl.BlockSpec((blk, TW), lambda i: (i, 0)),
        out_shape=jax.ShapeDtypeStruct((n_nodes, TW), jnp.float32),
    )(feat, W2, B)


def _compute_chunk(tbuf, efv, msgv):
    @pl.loop(0, CHUNK)
    def _(e):
        ef_row = efv[e, :]
        acc = [tbuf[e, pl.ds(F * F, F)], None, None, None]
        for d in range(F):
            t = ef_row[d] * tbuf[e, pl.ds(d * F, F)]
            k = d % 4
            acc[k] = t if acc[k] is None else acc[k] + t
        msgv[e, :] = (acc[0] + acc[1]) + (acc[2] + acc[3])


def _edge_chunks(base, nj, sidx, didx, taug_hbm, ef_hbm, acc_sh, deg_sh,
                 tbufs, efvs, msgs, onesv, gsems, esems, ssems, dsem):
    def fire(jj, b):
        pltpu.async_copy(ef_hbm.at[pl.ds(base + jj * CHUNK, CHUNK)],
                         efvs[b], esems[b])
        pltpu.async_copy(taug_hbm.at[sidx.at[jj]], tbufs[b], gsems[b])

    fire(0, 0)
    if nj > 1:
        fire(1, 1)

    @pl.loop(0, nj, step=2)
    def _(j):
        for b in range(2):
            jj = j + b

            @pl.when(jj < nj)
            def _():
                pltpu.make_async_copy(
                    ef_hbm.at[pl.ds(base + jj * CHUNK, CHUNK)],
                    efvs[b], esems[b]).wait()
                pltpu.make_async_copy(
                    taug_hbm.at[sidx.at[jj]], tbufs[b], gsems[b]).wait()

                @pl.when(jj >= 2)
                def _():
                    pltpu.make_async_copy(
                        msgs[b], acc_sh.at[didx.at[jj]], ssems[b]).wait()

                _compute_chunk(tbufs[b], efvs[b], msgs[b])

                @pl.when(jj + 2 < nj)
                def _():
                    fire(jj + 2, b)

                pltpu.async_copy(msgs[b], acc_sh.at[didx.at[jj]], ssems[b],
                                 add=True)
                pltpu.async_copy(onesv, deg_sh.at[didx.at[jj]], dsem,
                                 add=True)

    for k in range(min(2, nj)):
        b = (nj - 1 - k) % 2
        pltpu.make_async_copy(msgs[b], acc_sh.at[didx.at[0]], ssems[b]).wait()

    @pl.loop(0, nj)
    def _(j):
        pltpu.make_async_copy(onesv, deg_sh.at[didx.at[0]], dsem).wait()


def _load_idx(base, nj, ei_hbm, sidx, didx, isem):
    for j in range(nj):
        pltpu.async_copy(ei_hbm.at[0, pl.ds(base + j * CHUNK, CHUNK)],
                         sidx.at[j], isem)
        pltpu.async_copy(ei_hbm.at[1, pl.ds(base + j * CHUNK, CHUNK)],
                         didx.at[j], isem)
    for j in range(2 * nj):
        pltpu.make_async_copy(ei_hbm.at[0, pl.ds(base, CHUNK)],
                              sidx.at[0], isem).wait()


def _fused_body(kj, fullw, lastn, epw, n_acc, rps,
                taug_hbm, ef_hbm, ei_hbm, ones_hbm, zeros_hbm,
                acc_out, deg_out,
                acc_sh, deg_sh, sidx, didx, tb0, tb1, ef0, ef1, ms0, ms1,
                onesv, outv, gs0, gs1, es0, es1, ss0, ss1, dsem, isem):
    c = lax.axis_index("c")
    s = lax.axis_index("s")
    wid = s * NC + c
    base = wid * epw
    tbufs, efvs, msgs = (tb0, tb1), (ef0, ef1), (ms0, ms1)
    gsems, esems, ssems = (gs0, gs1), (es0, es1), (ss0, ss1)
    pltpu.sync_copy(zeros_hbm, acc_sh.at[pl.ds(s * rps, rps)])
    pltpu.sync_copy(zeros_hbm, deg_sh.at[pl.ds(s * rps, rps)])
    pltpu.sync_copy(ones_hbm, onesv)
    plsc.subcore_barrier()

    @pl.when(wid < fullw)
    def _():
        _load_idx(base, kj, ei_hbm, sidx, didx, isem)
        _edge_chunks(base, kj, sidx, didx, taug_hbm, ef_hbm, acc_sh, deg_sh,
                     tbufs, efvs, msgs, onesv, gsems, esems, ssems, dsem)

    if lastn:
        @pl.when(wid == fullw)
        def _():
            _load_idx(base, lastn, ei_hbm, sidx, didx, isem)
            _edge_chunks(base, lastn, sidx, didx, taug_hbm, ef_hbm, acc_sh,
                         deg_sh, tbufs, efvs, msgs, onesv, gsems, esems,
                         ssems, dsem)

    plsc.subcore_barrier()
    pltpu.sync_copy(acc_sh.at[pl.ds(s * rps, rps)], outv)
    pltpu.sync_copy(outv, acc_out.at[c, pl.ds(s * rps, rps)])
    pltpu.sync_copy(deg_sh.at[pl.ds(s * rps, rps)], outv)
    pltpu.sync_copy(outv, deg_out.at[c, pl.ds(s * rps, rps)])


def _sc_fused(taug, efeat, ei, n_acc, kj, fullw, lastn):
    epw = kj * CHUNK
    rps = n_acc // NS
    mesh = plsc.VectorSubcoreMesh(core_axis_name="c", subcore_axis_name="s")
    ones = jnp.ones((CHUNK, F), jnp.float32)
    zeros = jnp.zeros((rps, F), jnp.float32)
    out_sds = jax.ShapeDtypeStruct((NC, n_acc, F), jnp.float32)
    return pl.kernel(
        functools.partial(_fused_body, kj, fullw, lastn, epw, n_acc, rps),
        out_type=(out_sds, out_sds),
        mesh=mesh,
        scratch_types=[
            pltpu.VMEM_SHARED((n_acc, F), jnp.float32),
            pltpu.VMEM_SHARED((n_acc, F), jnp.float32),
            pltpu.VMEM((kj, CHUNK), jnp.int32),
            pltpu.VMEM((kj, CHUNK), jnp.int32),
            pltpu.VMEM((CHUNK, TW), jnp.float32),
            pltpu.VMEM((CHUNK, TW), jnp.float32),
            pltpu.VMEM((CHUNK, F), jnp.float32),
            pltpu.VMEM((CHUNK, F), jnp.float32),
            pltpu.VMEM((CHUNK, F), jnp.float32),
            pltpu.VMEM((CHUNK, F), jnp.float32),
            pltpu.VMEM((CHUNK, F), jnp.float32),
            pltpu.VMEM((rps, F), jnp.float32),
            pltpu.SemaphoreType.DMA,
            pltpu.SemaphoreType.DMA,
            pltpu.SemaphoreType.DMA,
            pltpu.SemaphoreType.DMA,
            pltpu.SemaphoreType.DMA,
            pltpu.SemaphoreType.DMA,
            pltpu.SemaphoreType.DMA,
            pltpu.SemaphoreType.DMA,
        ],
        compiler_params=pltpu.CompilerParams(use_tc_tiling_on_sc=False),
    )(taug, efeat, ei, ones, zeros)


def _final_body(n_nodes, rows, a_ref, d_ref, m_ref, bias_ref, g_ref, bt_ref,
                o_ref):
    acc = (a_ref[0] + a_ref[1])[:rows]
    deg = jnp.maximum((d_ref[0] + d_ref[1])[:rows], 1.0)
    rst = acc / deg + bias_ref[...]
    ssum = jnp.sum(rst, axis=0, keepdims=True)
    ssq = jnp.sum(rst * rst, axis=0, keepdims=True)
    mean = jnp.dot(ssum, m_ref[...], preferred_element_type=jnp.float32)
    ex2 = jnp.dot(ssq, m_ref[...], preferred_element_type=jnp.float32)
    var = ex2 - mean * mean
    inv = lax.rsqrt(var + 1e-5)
    o_ref[...] = (rst - mean) * inv * g_ref[...] + bt_ref[...]


def _tc_final(accf, degf, bias, gamma, beta, n_nodes, rows):
    lanes = 128
    per = lanes // F
    mf = ((jnp.arange(lanes)[:, None] % F)
          == (jnp.arange(lanes)[None, :] % F)).astype(jnp.float32) / n_nodes
    biasf = jnp.tile(bias, per)[None, :]
    gammaf = jnp.tile(gamma, per)[None, :]
    betaf = jnp.tile(beta, per)[None, :]
    return pl.pallas_call(
        functools.partial(_final_body, n_nodes, rows),
        out_shape=jax.ShapeDtypeStruct((rows, lanes), jnp.float32),
    )(accf, degf, mf, biasf, gammaf, betaf)


def kernel(feat, efeat, W_e, b_e, bias, gamma, beta, edge_index):
    n_nodes, in_f = feat.shape
    n_edges = edge_index.shape[1]
    out_f = bias.shape[0]

    n_acc = -(-(n_nodes + 1) // 128) * 128
    trash = n_nodes

    ei = edge_index
    if n_edges % CHUNK:
        pad = CHUNK - n_edges % CHUNK
        src = jnp.concatenate([ei[0], jnp.zeros((pad,), jnp.int32)])
        dst = jnp.concatenate([ei[1], jnp.full((pad,), trash, jnp.int32)])
        efeat = jnp.concatenate([efeat, jnp.zeros((pad, in_f), efeat.dtype)])
        ei = jnp.stack([src, dst])
        n_edges += pad
    nchunk = n_edges // CHUNK
    kj = -(-nchunk // NW)
    fullw = nchunk // kj
    lastn = nchunk - fullw * kj

    W2 = W_e.reshape(in_f, in_f, out_f).transpose(1, 0, 2).reshape(
        in_f, in_f * out_f)
    B = b_e.reshape(in_f, out_f)
    taug = _tc_pre(feat, W2, B, n_nodes, 2000)

    acc2, deg2 = _sc_fused(taug, efeat, ei, n_acc, kj, fullw, lastn)

    rows_acc = n_acc * out_f // 128
    rows = n_nodes * out_f // 128
    accf = acc2.reshape(NC, rows_acc, 128)
    degf = deg2.reshape(NC, rows_acc, 128)
    outf = _tc_final(accf, degf, bias, gamma, beta, n_nodes, rows)
    return outf.reshape(n_nodes, out_f)
